# baseline (speedup 1.0000x reference)
import functools

import jax
import jax.numpy as jnp
from jax import lax
from jax.experimental import pallas as pl
from jax.experimental.pallas import tpu as pltpu
from jax.experimental.pallas import tpu_sc as plsc

_BB = 8


def _dense_body(q_ref, lp_ref, eps_ref, chunk_ref, sub_ref):
    n = q_ref.shape[-1]
    q = q_ref[...]
    m = jnp.max(q, axis=-1, keepdims=True)
    idx = jnp.argmax(q, axis=-1).astype(jnp.int32)
    s = jnp.sum(jnp.exp(q - m), axis=-1, keepdims=True)
    lse = m + jnp.log(s)
    lp_ref[...] = q - lse
    eps_ref[...] = jnp.broadcast_to(m, (_BB, 128))
    row = pl.program_id(0) * _BB + lax.broadcasted_iota(jnp.int32, (_BB, 128), 0)
    flat = idx[:, None] + row * n
    chunk_ref[...] = flat >> 3
    sub_ref[...] = flat & 7


def _dense_call(q_mean):
    b, n = q_mean.shape
    return pl.pallas_call(
        _dense_body,
        grid=(b // _BB,),
        in_specs=[pl.BlockSpec((_BB, n), lambda i: (i, 0))],
        out_specs=[
            pl.BlockSpec((_BB, n), lambda i: (i, 0)),
            pl.BlockSpec((_BB, 128), lambda i: (i, 0)),
            pl.BlockSpec((_BB, 128), lambda i: (i, 0)),
            pl.BlockSpec((_BB, 128), lambda i: (i, 0)),
        ],
        out_shape=[
            jax.ShapeDtypeStruct((b, n), jnp.float32),
            jax.ShapeDtypeStruct((b, 128), jnp.float32),
            jax.ShapeDtypeStruct((b, 128), jnp.int32),
            jax.ShapeDtypeStruct((b, 128), jnp.int32),
        ],
    )(q_mean)


def _sc_gather_call(table, chunk_ind, sub_ind):
    b = chunk_ind.shape[0]
    a = table.shape[-1]
    n_workers = 16
    rpw = b // n_workers
    mesh = plsc.VectorSubcoreMesh(core_axis_name="c", subcore_axis_name="s")

    @functools.partial(
        pl.kernel,
        out_type=jax.ShapeDtypeStruct((b, a), jnp.float32),
        mesh=mesh,
        compiler_params=pltpu.CompilerParams(use_tc_tiling_on_sc=False),
        scratch_types=[
            pltpu.VMEM((rpw,), jnp.int32),
            pltpu.VMEM((16,), jnp.int32),
            pltpu.VMEM((rpw, 8, a), jnp.float32),
            pltpu.VMEM((rpw, a), jnp.float32),
            pltpu.SemaphoreType.DMA,
        ],
    )
    def gather_kernel(table_hbm, chunk_hbm, sub_hbm, out_hbm,
                      cidx_v, sidx_v, rows_v, final_v, sem):
        wid = lax.axis_index("s") * 2 + lax.axis_index("c")

        @pl.when(wid < n_workers)
        def _():
            base = wid * rpw
            pltpu.sync_copy(chunk_hbm.at[pl.ds(base, rpw)], cidx_v)
            pltpu.sync_copy(sub_hbm.at[pl.ds(base, rpw)], sidx_v.at[pl.ds(0, rpw)])
            pltpu.async_copy(table_hbm.at[cidx_v], rows_v, sem).wait()
            sub = sidx_v[...]
            for r in range(rpw):
                j = sub[r]
                final_v[r, :] = rows_v[r, j, :]
            pltpu.sync_copy(final_v, out_hbm.at[pl.ds(base, rpw)])

    return gather_kernel(table, chunk_ind, sub_ind)


def kernel(q_mean, q_stddev, actions):
    del q_stddev
    b, n, a = actions.shape
    log_probs, eps, chunk, sub = _dense_call(q_mean)
    best_eps = eps[:, 0]
    table = actions.reshape(b * n // 8, 8, a)
    best_action = _sc_gather_call(table, chunk[:, 0], sub[:, 0])
    return (log_probs, best_action, best_eps)


# speedup vs baseline: 5.6770x; 5.6770x over previous
import functools

import jax
import jax.numpy as jnp
from jax import lax
from jax.experimental import pallas as pl
from jax.experimental.pallas import tpu as pltpu
from jax.experimental.pallas import tpu_sc as plsc

_BB = 8


def _dense_body(q_ref, lp_ref, eps_ref, chunk_ref, sub_ref):
    n = q_ref.shape[-1]
    q = q_ref[...]
    m = jnp.max(q, axis=-1, keepdims=True)
    idx = jnp.argmax(q, axis=-1).astype(jnp.int32)
    s = jnp.sum(jnp.exp(q - m), axis=-1, keepdims=True)
    lse = m + jnp.log(s)
    lp_ref[...] = q - lse
    eps_ref[...] = jnp.broadcast_to(m, (_BB, 128))
    row = pl.program_id(0) * _BB + lax.broadcasted_iota(jnp.int32, (_BB, 128), 0)
    flat = idx[:, None] + row * n
    chunk_ref[...] = flat >> 3
    sub_ref[...] = flat & 7


def _dense_call(q_mean):
    b, n = q_mean.shape
    return pl.pallas_call(
        _dense_body,
        grid=(b // _BB,),
        in_specs=[pl.BlockSpec((_BB, n), lambda i: (i, 0))],
        out_specs=[
            pl.BlockSpec((_BB, n), lambda i: (i, 0)),
            pl.BlockSpec((_BB, 128), lambda i: (i, 0)),
            pl.BlockSpec((_BB, 128), lambda i: (i, 0)),
            pl.BlockSpec((_BB, 128), lambda i: (i, 0)),
        ],
        out_shape=[
            jax.ShapeDtypeStruct((b, n), jnp.float32),
            jax.ShapeDtypeStruct((b, 128), jnp.float32),
            jax.ShapeDtypeStruct((b, 128), jnp.int32),
            jax.ShapeDtypeStruct((b, 128), jnp.int32),
        ],
    )(q_mean)


def _sc_gather_call(table, chunk_ind, sub_ind):
    b = chunk_ind.shape[0]
    a = table.shape[-1]
    n_workers = 16
    rpw = b // n_workers
    mesh = plsc.VectorSubcoreMesh(core_axis_name="c", subcore_axis_name="s")

    @functools.partial(
        pl.kernel,
        out_type=jax.ShapeDtypeStruct((b, a), jnp.float32),
        mesh=mesh,
        scratch_types=[
            pltpu.VMEM((16,), jnp.int32),
            pltpu.VMEM((16,), jnp.int32),
            pltpu.VMEM((rpw, 8, a), jnp.float32),
            pltpu.VMEM((rpw, a), jnp.float32),
            pltpu.SemaphoreType.DMA,
        ],
    )
    def gather_kernel(table_hbm, chunk_hbm, sub_hbm, out_hbm,
                      cidx_v, sidx_v, rows_v, final_v, sem):
        wid = lax.axis_index("s") * 2 + lax.axis_index("c")

        @pl.when(wid < n_workers)
        def _():
            base = wid * rpw
            pltpu.sync_copy(chunk_hbm.at[pl.ds(base, rpw)], cidx_v.at[pl.ds(0, rpw)])
            pltpu.sync_copy(sub_hbm.at[pl.ds(base, rpw)], sidx_v.at[pl.ds(0, rpw)])
            cv = cidx_v[...]
            sub = sidx_v[...]
            copies = []
            for r in range(rpw):
                cp = pltpu.make_async_copy(
                    table_hbm.at[cv[r]], rows_v.at[r], sem)
                cp.start()
                copies.append(cp)
            for cp in copies:
                cp.wait()
            for r in range(rpw):
                j = sub[r]
                final_v[r, :] = rows_v[r, j, :]
            pltpu.sync_copy(final_v, out_hbm.at[pl.ds(base, rpw)])

    return gather_kernel(table, chunk_ind, sub_ind)


def kernel(q_mean, q_stddev, actions):
    del q_stddev
    b, n, a = actions.shape
    log_probs, eps, chunk, sub = _dense_call(q_mean)
    best_eps = eps[:, 0]
    table = actions.reshape(b * n // 8, 8, a)
    best_action = _sc_gather_call(table, chunk[:, 0], sub[:, 0])
    return (log_probs, best_action, best_eps)
